# baseline (speedup 1.0000x reference)
import functools

import jax
import jax.numpy as jnp
from jax import lax
from jax.experimental import pallas as pl
from jax.experimental.pallas import tpu as pltpu
from jax.experimental.pallas import tpu_sc as plsc

_K = 1024
_D = 50257
_B = 8
_KNN = 8
_NCLASS = 2
_SCALE = -1.0 / (0.05 * _D)
_BK = 64
_L = 16


_BD = 4096
_NB = -(-_D // _BD)


def _dist_body(logits_ref, anchor_t_ref, out_ref, alam_ref):
    i = pl.program_id(0)

    @pl.when(i == 0)
    def _():
        out_ref[...] = jnp.zeros_like(out_ref)
        alam_ref[...] = jnp.zeros_like(alam_ref)

    def step(masked):
        a = anchor_t_ref[...]
        ll = jnp.log(logits_ref[...])
        if masked:
            jrow = lax.broadcasted_iota(jnp.int32, (_BD, _K), 0) + i * _BD
            a = jnp.where(jrow < _D, a, 1.0)
            jcol = lax.broadcasted_iota(jnp.int32, (_B, _BD), 1) + i * _BD
            ll = jnp.where(jcol < _D, ll, 0.0)
        la = jnp.log(a)
        s_bf = (a * la).astype(jnp.bfloat16)
        a_bf = a.astype(jnp.bfloat16)
        ll_bf = ll.astype(jnp.bfloat16)
        out_ref[...] += lax.dot_general(
            ll_bf, a_bf, (((1,), (0,)), ((), ())),
            preferred_element_type=jnp.float32)
        ones = jnp.ones((1, _BD), jnp.bfloat16)
        alam_ref[...] += lax.dot_general(
            ones, s_bf, (((1,), (0,)), ((), ())),
            preferred_element_type=jnp.float32)

    @pl.when(i < _NB - 1)
    def _():
        step(masked=False)

    @pl.when(i == _NB - 1)
    def _():
        step(masked=True)
        out_ref[...] = _SCALE * (alam_ref[...] - out_ref[...])


def _scaled_dists(logits, queue_anchor_t, interpret=False):
    return pl.pallas_call(
        _dist_body,
        grid=(_NB,),
        in_specs=[
            pl.BlockSpec((_B, _BD), lambda i: (0, i)),
            pl.BlockSpec((_BD, _K), lambda i: (i, 0)),
        ],
        out_specs=pl.BlockSpec((_B, _K), lambda i: (0, 0)),
        out_shape=jax.ShapeDtypeStruct((_B, _K), jnp.float32),
        scratch_shapes=[pltpu.VMEM((1, _K), jnp.float32)],
        interpret=interpret,
    )(logits, queue_anchor_t)


def _sc_topk_body(dists, labels, out, row_v, lab_v, res_v):
    c = lax.axis_index("c")
    s = lax.axis_index("s")
    wid = s * 2 + c

    @pl.when(wid < _B)
    def _():
        pltpu.sync_copy(dists.at[wid], row_v)
        pltpu.sync_copy(labels, lab_v)
        lane = lax.iota(jnp.int32, _L)

        def body(v, carry):
            tv, ti = carry
            vals = row_v[pl.ds(v * _L, _L)]
            idx = lane + v * _L
            sv, si = plsc.sort_key_val(vals, idx, descending=False)
            m = tv >= sv
            nv = jnp.where(m, tv, sv)
            ni = jnp.where(m, ti, si)
            nv, ni = plsc.sort_key_val(nv, ni, descending=True)
            return nv, ni

        tv, ti = lax.fori_loop(
            0, _K // _L, body,
            (jnp.full((_L,), -jnp.inf, jnp.float32),
             jnp.zeros((_L,), jnp.int32)))

        w = jnp.exp(tv - jnp.max(tv))
        w = jnp.where(lane < _KNN, w, 0.0)
        labg = plsc.load_gather(lab_v, [ti])
        denom = jnp.sum(w)
        wn = w / jnp.broadcast_to(denom, (_L,))
        p1 = jnp.sum(wn * labg)
        res_v[...] = jnp.where(lane == 0, 1.0 - p1,
                               jnp.where(lane == 1, p1, 0.0))
        pltpu.sync_copy(res_v, out.at[wid])


@functools.cache
def _sc_topk():
    return pl.kernel(
        _sc_topk_body,
        out_type=jax.ShapeDtypeStruct((_B, _L), jnp.float32),
        mesh=plsc.VectorSubcoreMesh(core_axis_name="c", subcore_axis_name="s"),
        compiler_params=pltpu.CompilerParams(needs_layout_passes=False),
        scratch_types=[
            pltpu.VMEM((_K,), jnp.float32),
            pltpu.VMEM((_K,), jnp.float32),
            pltpu.VMEM((_L,), jnp.float32),
        ],
    )


def kernel(logits, queue_anchor, queue_label):
    dists = _scaled_dists(logits, queue_anchor.T)
    labf = queue_label.astype(jnp.float32)
    out16 = _sc_topk()(dists, labf)
    return out16[:, :_NCLASS]


# speedup vs baseline: 1.0930x; 1.0930x over previous
import functools

import jax
import jax.numpy as jnp
from jax import lax
from jax.experimental import pallas as pl
from jax.experimental.pallas import tpu as pltpu
from jax.experimental.pallas import tpu_sc as plsc

_K = 1024
_D = 50257
_B = 8
_KNN = 8
_NCLASS = 2
_SCALE = -1.0 / (0.05 * _D)
_BK = 64
_L = 16


_BD = 4096
_NB = -(-_D // _BD)


def _dist_body(logits_ref, anchor_t_ref, out_ref, alam_ref):
    i = pl.program_id(0)

    @pl.when(i == 0)
    def _():
        out_ref[...] = jnp.zeros_like(out_ref)
        alam_ref[...] = jnp.zeros_like(alam_ref)

    def step(masked):
        a = anchor_t_ref[...]
        ll = jnp.log(logits_ref[...])
        if masked:
            jrow = lax.broadcasted_iota(jnp.int32, (_BD, _K), 0) + i * _BD
            a = jnp.where(jrow < _D, a, 1.0)
            jcol = lax.broadcasted_iota(jnp.int32, (_B, _BD), 1) + i * _BD
            ll = jnp.where(jcol < _D, ll, 0.0)
        la = jnp.log(a)
        alam_ref[...] += jnp.sum(a * la, axis=0, keepdims=True)
        a_bf = a.astype(jnp.bfloat16)
        ll_bf = ll.astype(jnp.bfloat16)
        out_ref[...] += lax.dot_general(
            ll_bf, a_bf, (((1,), (0,)), ((), ())),
            preferred_element_type=jnp.float32)

    @pl.when(i < _NB - 1)
    def _():
        step(masked=False)

    @pl.when(i == _NB - 1)
    def _():
        step(masked=True)
        out_ref[...] = _SCALE * (alam_ref[...] - out_ref[...])


def _scaled_dists(logits, queue_anchor_t, interpret=False):
    return pl.pallas_call(
        _dist_body,
        grid=(_NB,),
        in_specs=[
            pl.BlockSpec((_B, _BD), lambda i: (0, i)),
            pl.BlockSpec((_BD, _K), lambda i: (i, 0)),
        ],
        out_specs=pl.BlockSpec((_B, _K), lambda i: (0, 0)),
        out_shape=jax.ShapeDtypeStruct((_B, _K), jnp.float32),
        scratch_shapes=[pltpu.VMEM((1, _K), jnp.float32)],
        interpret=interpret,
    )(logits, queue_anchor_t)


def _sc_topk_body(dists, labels, out, row_v, lab_v, res_v):
    c = lax.axis_index("c")
    s = lax.axis_index("s")
    wid = s * 2 + c

    @pl.when(wid < _B)
    def _():
        pltpu.sync_copy(dists.at[wid], row_v)
        pltpu.sync_copy(labels, lab_v)
        lane = lax.iota(jnp.int32, _L)

        def body(v, carry):
            tv, ti = carry
            vals = row_v[pl.ds(v * _L, _L)]
            idx = lane + v * _L
            sv, si = plsc.sort_key_val(vals, idx, descending=False)
            m = tv >= sv
            nv = jnp.where(m, tv, sv)
            ni = jnp.where(m, ti, si)
            nv, ni = plsc.sort_key_val(nv, ni, descending=True)
            return nv, ni

        tv, ti = lax.fori_loop(
            0, _K // _L, body,
            (jnp.full((_L,), -jnp.inf, jnp.float32),
             jnp.zeros((_L,), jnp.int32)))

        w = jnp.exp(tv - jnp.max(tv))
        w = jnp.where(lane < _KNN, w, 0.0)
        labg = plsc.load_gather(lab_v, [ti])
        denom = jnp.sum(w)
        wn = w / jnp.broadcast_to(denom, (_L,))
        p1 = jnp.sum(wn * labg)
        res_v[...] = jnp.where(lane == 0, 1.0 - p1,
                               jnp.where(lane == 1, p1, 0.0))
        pltpu.sync_copy(res_v, out.at[wid])


@functools.cache
def _sc_topk():
    return pl.kernel(
        _sc_topk_body,
        out_type=jax.ShapeDtypeStruct((_B, _L), jnp.float32),
        mesh=plsc.VectorSubcoreMesh(core_axis_name="c", subcore_axis_name="s"),
        compiler_params=pltpu.CompilerParams(needs_layout_passes=False),
        scratch_types=[
            pltpu.VMEM((_K,), jnp.float32),
            pltpu.VMEM((_K,), jnp.float32),
            pltpu.VMEM((_L,), jnp.float32),
        ],
    )


def kernel(logits, queue_anchor, queue_label):
    dists = _scaled_dists(logits, queue_anchor.T)
    labf = queue_label.astype(jnp.float32)
    out16 = _sc_topk()(dists, labf)
    return out16[:, :_NCLASS]


# speedup vs baseline: 1.1341x; 1.0376x over previous
import functools

import jax
import jax.numpy as jnp
from jax import lax
from jax.experimental import pallas as pl
from jax.experimental.pallas import tpu as pltpu
from jax.experimental.pallas import tpu_sc as plsc

_K = 1024
_D = 50257
_B = 8
_KNN = 8
_NCLASS = 2
_SCALE = -1.0 / (0.05 * _D)
_BK = 64
_L = 16


_BD = 4096
_NB = -(-_D // _BD)


_BH = _BD // 2
_LASTB = (_D - 1) // _BH


def _dist_body(logits_ref, a0_ref, a1_ref, out_ref, alam_ref):
    i = pl.program_id(0)

    @pl.when(i == 0)
    def _():
        out_ref[...] = jnp.zeros_like(out_ref)
        alam_ref[...] = jnp.zeros_like(alam_ref)

    ll = jnp.log(logits_ref[...])
    jcol = lax.broadcasted_iota(jnp.int32, (_B, _BD), 1) + i * _BD
    ll = jnp.where(jcol < _D, ll, 0.0).astype(jnp.bfloat16)

    def half(aref, b, llh):
        a = aref[...]
        jrow = lax.broadcasted_iota(jnp.int32, (_BH, _K), 0) + b * _BH
        a = jnp.where(jrow < _D, a, 1.0)
        la = jnp.log(a)
        alam_ref[...] += jnp.sum(a * la, axis=0, keepdims=True)
        out_ref[...] += lax.dot_general(
            llh, a.astype(jnp.bfloat16), (((1,), (0,)), ((), ())),
            preferred_element_type=jnp.float32)

    half(a0_ref, 2 * i, ll[:, :_BH])
    half(a1_ref, 2 * i + 1, ll[:, _BH:])

    @pl.when(i == _NB - 1)
    def _():
        out_ref[...] = _SCALE * (alam_ref[...] - out_ref[...])


def _scaled_dists(logits, queue_anchor_t, interpret=False):
    return pl.pallas_call(
        _dist_body,
        grid=(_NB,),
        in_specs=[
            pl.BlockSpec((_B, _BD), lambda i: (0, i)),
            pl.BlockSpec((_BH, _K), lambda i: (2 * i, 0)),
            pl.BlockSpec((_BH, _K),
                         lambda i: (jnp.minimum(2 * i + 1, _LASTB), 0)),
        ],
        out_specs=pl.BlockSpec((_B, _K), lambda i: (0, 0)),
        out_shape=jax.ShapeDtypeStruct((_B, _K), jnp.float32),
        scratch_shapes=[pltpu.VMEM((1, _K), jnp.float32)],
        interpret=interpret,
    )(logits, queue_anchor_t, queue_anchor_t)


def _sc_topk_body(dists, labels, out, row_v, lab_v, res_v):
    c = lax.axis_index("c")
    s = lax.axis_index("s")
    wid = s * 2 + c

    @pl.when(wid < _B)
    def _():
        pltpu.sync_copy(dists.at[wid], row_v)
        pltpu.sync_copy(labels, lab_v)
        lane = lax.iota(jnp.int32, _L)

        def body(v, carry):
            tv, ti = carry
            vals = row_v[pl.ds(v * _L, _L)]
            idx = lane + v * _L
            sv, si = plsc.sort_key_val(vals, idx, descending=False)
            m = tv >= sv
            nv = jnp.where(m, tv, sv)
            ni = jnp.where(m, ti, si)
            nv, ni = plsc.sort_key_val(nv, ni, descending=True)
            return nv, ni

        tv, ti = lax.fori_loop(
            0, _K // _L, body,
            (jnp.full((_L,), -jnp.inf, jnp.float32),
             jnp.zeros((_L,), jnp.int32)))

        w = jnp.exp(tv - jnp.max(tv))
        w = jnp.where(lane < _KNN, w, 0.0)
        labg = plsc.load_gather(lab_v, [ti])
        denom = jnp.sum(w)
        wn = w / jnp.broadcast_to(denom, (_L,))
        p1 = jnp.sum(wn * labg)
        res_v[...] = jnp.where(lane == 0, 1.0 - p1,
                               jnp.where(lane == 1, p1, 0.0))
        pltpu.sync_copy(res_v, out.at[wid])


@functools.cache
def _sc_topk():
    return pl.kernel(
        _sc_topk_body,
        out_type=jax.ShapeDtypeStruct((_B, _L), jnp.float32),
        mesh=plsc.VectorSubcoreMesh(core_axis_name="c", subcore_axis_name="s"),
        compiler_params=pltpu.CompilerParams(needs_layout_passes=False),
        scratch_types=[
            pltpu.VMEM((_K,), jnp.float32),
            pltpu.VMEM((_K,), jnp.float32),
            pltpu.VMEM((_L,), jnp.float32),
        ],
    )


def kernel(logits, queue_anchor, queue_label):
    dists = _scaled_dists(logits, queue_anchor.T)
    labf = queue_label.astype(jnp.float32)
    out16 = _sc_topk()(dists, labf)
    return out16[:, :_NCLASS]


# speedup vs baseline: 1.1383x; 1.0037x over previous
import functools

import jax
import jax.numpy as jnp
from jax import lax
from jax.experimental import pallas as pl
from jax.experimental.pallas import tpu as pltpu
from jax.experimental.pallas import tpu_sc as plsc

_K = 1024
_D = 50257
_B = 8
_KNN = 8
_NCLASS = 2
_SCALE = -1.0 / (0.05 * _D)
_BK = 64
_L = 16


_BD = 5120
_NB = -(-_D // _BD)


_BH = _BD // 2
_LASTB = (_D - 1) // _BH


def _dist_body(logits_ref, a0_ref, a1_ref, out_ref, alam_ref):
    i = pl.program_id(0)

    @pl.when(i == 0)
    def _():
        out_ref[...] = jnp.zeros_like(out_ref)
        alam_ref[...] = jnp.zeros_like(alam_ref)

    ll = jnp.log(logits_ref[...])
    jcol = lax.broadcasted_iota(jnp.int32, (_B, _BD), 1) + i * _BD
    ll = jnp.where(jcol < _D, ll, 0.0).astype(jnp.bfloat16)

    def half(aref, b, llh):
        a = aref[...]
        jrow = lax.broadcasted_iota(jnp.int32, (_BH, _K), 0) + b * _BH
        a = jnp.where(jrow < _D, a, 1.0)
        la = jnp.log(a)
        alam_ref[...] += jnp.sum(a * la, axis=0, keepdims=True)
        out_ref[...] += lax.dot_general(
            llh, a.astype(jnp.bfloat16), (((1,), (0,)), ((), ())),
            preferred_element_type=jnp.float32)

    half(a0_ref, 2 * i, ll[:, :_BH])
    half(a1_ref, 2 * i + 1, ll[:, _BH:])

    @pl.when(i == _NB - 1)
    def _():
        out_ref[...] = _SCALE * (alam_ref[...] - out_ref[...])


def _scaled_dists(logits, queue_anchor_t, interpret=False):
    return pl.pallas_call(
        _dist_body,
        grid=(_NB,),
        in_specs=[
            pl.BlockSpec((_B, _BD), lambda i: (0, i)),
            pl.BlockSpec((_BH, _K), lambda i: (2 * i, 0)),
            pl.BlockSpec((_BH, _K),
                         lambda i: (jnp.minimum(2 * i + 1, _LASTB), 0)),
        ],
        out_specs=pl.BlockSpec((_B, _K), lambda i: (0, 0)),
        out_shape=jax.ShapeDtypeStruct((_B, _K), jnp.float32),
        scratch_shapes=[pltpu.VMEM((1, _K), jnp.float32)],
        interpret=interpret,
    )(logits, queue_anchor_t, queue_anchor_t)


def _sc_topk_body(dists, labels, out, row_v, lab_v, res_v):
    c = lax.axis_index("c")
    s = lax.axis_index("s")
    wid = s * 2 + c

    @pl.when(wid < _B)
    def _():
        pltpu.sync_copy(dists.at[wid], row_v)
        pltpu.sync_copy(labels, lab_v)
        lane = lax.iota(jnp.int32, _L)

        def body(v, carry):
            tv, ti = carry
            vals = row_v[pl.ds(v * _L, _L)]
            idx = lane + v * _L
            sv, si = plsc.sort_key_val(vals, idx, descending=False)
            m = tv >= sv
            nv = jnp.where(m, tv, sv)
            ni = jnp.where(m, ti, si)
            nv, ni = plsc.sort_key_val(nv, ni, descending=True)
            return nv, ni

        tv, ti = lax.fori_loop(
            0, _K // _L, body,
            (jnp.full((_L,), -jnp.inf, jnp.float32),
             jnp.zeros((_L,), jnp.int32)))

        w = jnp.exp(tv - jnp.max(tv))
        w = jnp.where(lane < _KNN, w, 0.0)
        labg = plsc.load_gather(lab_v, [ti])
        denom = jnp.sum(w)
        wn = w / jnp.broadcast_to(denom, (_L,))
        p1 = jnp.sum(wn * labg)
        res_v[...] = jnp.where(lane == 0, 1.0 - p1,
                               jnp.where(lane == 1, p1, 0.0))
        pltpu.sync_copy(res_v, out.at[wid])


@functools.cache
def _sc_topk():
    return pl.kernel(
        _sc_topk_body,
        out_type=jax.ShapeDtypeStruct((_B, _L), jnp.float32),
        mesh=plsc.VectorSubcoreMesh(core_axis_name="c", subcore_axis_name="s"),
        compiler_params=pltpu.CompilerParams(needs_layout_passes=False),
        scratch_types=[
            pltpu.VMEM((_K,), jnp.float32),
            pltpu.VMEM((_K,), jnp.float32),
            pltpu.VMEM((_L,), jnp.float32),
        ],
    )


def kernel(logits, queue_anchor, queue_label):
    dists = _scaled_dists(logits, queue_anchor.T)
    labf = queue_label.astype(jnp.float32)
    out16 = _sc_topk()(dists, labf)
    return out16[:, :_NCLASS]
